# Initial kernel scaffold; baseline (speedup 1.0000x reference)
import functools

import jax
import jax.numpy as jnp
from jax import lax
from jax.experimental import pallas as pl
from jax.experimental.pallas import tpu as pltpu
from jax.experimental.pallas import tpu_sc as plsc

N = 10000
D = 128
NC, NS, L = 2, 16, 16
NW = NC * NS
B = 128
CHUNKS = 79
EPT = B * CHUNKS
E_PAD = EPT * NW
N_PAD = N + 16
N_SH = 10240

R = 400
G = N // R


def _tc_head_body(x_ref, w_ref, asv_ref, adv_ref, h_ref, as_ref, ad_ref):
    h = jnp.dot(x_ref[...], w_ref[...], preferred_element_type=jnp.float32)
    h_ref[...] = h
    as_ref[...] = jnp.sum(h * asv_ref[...], axis=1)[None, :]
    ad_ref[...] = jnp.sum(h * adv_ref[...], axis=1)[None, :]


def _tc_head(x, W, a_src, a_dst):
    return pl.pallas_call(
        _tc_head_body,
        grid=(G,),
        in_specs=[
            pl.BlockSpec((R, D), lambda i: (i, 0)),
            pl.BlockSpec((D, D), lambda i: (0, 0)),
            pl.BlockSpec((1, D), lambda i: (0, 0)),
            pl.BlockSpec((1, D), lambda i: (0, 0)),
        ],
        out_specs=[
            pl.BlockSpec((R, D), lambda i: (i, 0)),
            pl.BlockSpec((1, R), lambda i: (i, 0)),
            pl.BlockSpec((1, R), lambda i: (i, 0)),
        ],
        out_shape=[
            jax.ShapeDtypeStruct((N, D), jnp.float32),
            jax.ShapeDtypeStruct((G, R), jnp.float32),
            jax.ShapeDtypeStruct((G, R), jnp.float32),
        ],
    )(x, W, a_src.reshape(1, D), a_dst.reshape(1, D))


def _tc_mid_body(num_ref, den_ref, b_ref, w_ref, asv_ref, adv_ref,
                 h_ref, as_ref, ad_ref):
    num = num_ref[0] + num_ref[1]
    den = den_ref[0] + den_ref[1]
    y = num / (den + 1e-16)[:, None] + b_ref[...]
    h = jnp.dot(y, w_ref[...], preferred_element_type=jnp.float32)
    h_ref[...] = h
    as_ref[...] = jnp.sum(h * asv_ref[...], axis=1)[None, :]
    ad_ref[...] = jnp.sum(h * adv_ref[...], axis=1)[None, :]


def _tc_mid(num, den, b, W, a_src, a_dst):
    return pl.pallas_call(
        _tc_mid_body,
        grid=(G,),
        in_specs=[
            pl.BlockSpec((2, R, D), lambda i: (0, i, 0)),
            pl.BlockSpec((2, R), lambda i: (0, i)),
            pl.BlockSpec((1, D), lambda i: (0, 0)),
            pl.BlockSpec((D, D), lambda i: (0, 0)),
            pl.BlockSpec((1, D), lambda i: (0, 0)),
            pl.BlockSpec((1, D), lambda i: (0, 0)),
        ],
        out_specs=[
            pl.BlockSpec((R, D), lambda i: (i, 0)),
            pl.BlockSpec((1, R), lambda i: (i, 0)),
            pl.BlockSpec((1, R), lambda i: (i, 0)),
        ],
        out_shape=[
            jax.ShapeDtypeStruct((N, D), jnp.float32),
            jax.ShapeDtypeStruct((G, R), jnp.float32),
            jax.ShapeDtypeStruct((G, R), jnp.float32),
        ],
    )(num, den, b.reshape(1, D), W, a_src.reshape(1, D), a_dst.reshape(1, D))


def _tc_fin_body(num_ref, den_ref, b_ref, o_ref):
    num = num_ref[0] + num_ref[1]
    den = den_ref[0] + den_ref[1]
    y = num / (den + 1e-16)[:, None] + b_ref[...]
    o_ref[...] = jnp.maximum(y, 0.0)


def _tc_fin(num, den, b):
    return pl.pallas_call(
        _tc_fin_body,
        grid=(G,),
        in_specs=[
            pl.BlockSpec((2, R, D), lambda i: (0, i, 0)),
            pl.BlockSpec((2, R), lambda i: (0, i)),
            pl.BlockSpec((1, D), lambda i: (0, 0)),
        ],
        out_specs=pl.BlockSpec((R, D), lambda i: (i, 0)),
        out_shape=jax.ShapeDtypeStruct((N, D), jnp.float32),
    )(num, den, b.reshape(1, D))


def _sc_edge_body(h_hbm, src_hbm, dst_hbm, asrc_hbm, adst_hbm,
                  num_out, den_out,
                  asrc_v, adst_v, sidx_v, didx_v, ex_v, rows_v,
                  num_sh, den_sh, sem):
    c = lax.axis_index("c")
    s = lax.axis_index("s")
    wid = c * NS + s

    pltpu.sync_copy(asrc_hbm, asrc_v)
    pltpu.sync_copy(adst_hbm, adst_v)

    init = (jnp.full((L,), -1e30, jnp.float32),
            jnp.full((L,), -1e30, jnp.float32))

    @pl.loop(0, N_PAD // L, init_carry=init)
    def _maxloop(i, carry):
        ma, md = carry
        sl = pl.ds(i * L, L)
        return (jnp.maximum(ma, asrc_v[sl]), jnp.maximum(md, adst_v[sl]))

    ma, md = _maxloop
    csum = jnp.max(ma) + jnp.max(md)
    cub = jnp.maximum(csum, 0.2 * csum)

    zv = jnp.zeros((L,), jnp.float32)

    @pl.loop(0, B)
    def _zrows(ei):
        for j in range(D // L):
            rows_v[ei, pl.ds(j * L, L)] = zv

    for j in range(B // L):
        ex_v[pl.ds(j * L, L)] = zv

    for k in range(5):
        pltpu.sync_copy(rows_v, num_sh.at[pl.ds(s * 640 + k * B, B)])
    for k in range(5):
        pltpu.sync_copy(ex_v, den_sh.at[pl.ds(s * 640 + k * B, B)])
    plsc.subcore_barrier()

    @pl.loop(0, CHUNKS)
    def _chunk(ch):
        base = wid * EPT + ch * B
        pltpu.sync_copy(src_hbm.at[pl.ds(base, B)], sidx_v)
        pltpu.sync_copy(dst_hbm.at[pl.ds(base, B)], didx_v)
        pltpu.async_copy(h_hbm.at[sidx_v], rows_v, sem).wait()
        for j in range(B // L):
            sl = pl.ds(j * L, L)
            si = sidx_v[sl]
            di = didx_v[sl]
            al = plsc.load_gather(asrc_v, [si]) + plsc.load_gather(adst_v, [di])
            al = jnp.maximum(al, 0.2 * al)
            ex_v[sl] = jnp.exp(al - cub)

        @pl.loop(0, B)
        def _scale(ei):
            coef = ex_v[ei]
            for j in range(D // L):
                sl = pl.ds(j * L, L)
                rows_v[ei, sl] = rows_v[ei, sl] * coef

        pltpu.sync_copy(rows_v, num_sh.at[didx_v], add=True)
        pltpu.sync_copy(ex_v, den_sh.at[didx_v], add=True)

    plsc.subcore_barrier()

    for k in range(5):
        sl = pl.ds(s * 640 + k * B, B)
        pltpu.sync_copy(num_sh.at[sl], num_out.at[c, sl])
    pltpu.sync_copy(den_sh.at[pl.ds(s * 640, 640)],
                    den_out.at[c, pl.ds(s * 640, 640)])


_sc_edge = pl.kernel(
    _sc_edge_body,
    out_type=(
        jax.ShapeDtypeStruct((NC, N_SH, D), jnp.float32),
        jax.ShapeDtypeStruct((NC, N_SH), jnp.float32),
    ),
    mesh=plsc.VectorSubcoreMesh(core_axis_name="c", subcore_axis_name="s",
                                num_cores=NC, num_subcores=NS),
    scratch_types=[
        pltpu.VMEM((N_PAD,), jnp.float32),
        pltpu.VMEM((N_PAD,), jnp.float32),
        pltpu.VMEM((B,), jnp.int32),
        pltpu.VMEM((B,), jnp.int32),
        pltpu.VMEM((B,), jnp.float32),
        pltpu.VMEM((B, D), jnp.float32),
        pltpu.VMEM_SHARED((N_SH, D), jnp.float32),
        pltpu.VMEM_SHARED((N_SH,), jnp.float32),
        pltpu.SemaphoreType.DMA,
    ],
)


def kernel(x, e, W1, a_src1, a_dst1, b1, W2, a_src2, a_dst2, b2):
    src = e[0].astype(jnp.int32)
    dst = e[1].astype(jnp.int32)
    pad = E_PAD - src.shape[0]
    src_p = jnp.concatenate([src, jnp.full((pad,), N, jnp.int32)])
    dst_p = jnp.concatenate([dst, jnp.full((pad,), N, jnp.int32)])

    def layer_edge(h, a_s, a_d):
        hp = jnp.pad(h, ((0, N_PAD - N), (0, 0)))
        asp = jnp.pad(a_s.reshape(-1), (0, N_PAD - N))
        adp = jnp.pad(a_d.reshape(-1), (0, N_PAD - N))
        return _sc_edge(hp, src_p, dst_p, asp, adp)

    h1, as1, ad1 = _tc_head(x, W1, a_src1, a_dst1)
    num1, den1 = layer_edge(h1, as1, ad1)
    h2, as2, ad2 = _tc_mid(num1[:, :N, :], den1, b1, W2, a_src2, a_dst2)
    num2, den2 = layer_edge(h2, as2, ad2)
    return _tc_fin(num2[:, :N, :], den2, b2)


# speedup vs baseline: 20.0884x; 20.0884x over previous
import jax
import jax.numpy as jnp
from jax import lax
from jax.experimental import pallas as pl
from jax.experimental.pallas import tpu as pltpu
from jax.experimental.pallas import tpu_sc as plsc

N = 10000
D = 128
NT = 10240
NC, NS, L = 2, 16, 16
NW = NC * NS
B = 128
CHUNKS = 79
EPT = B * CHUNKS
E_PAD = EPT * NW

R = 512
G = NT // R


def _tc_head_body(x_ref, w_ref, asv_ref, adv_ref, h_ref, as_ref, ad_ref):
    h = jnp.dot(x_ref[...], w_ref[...], preferred_element_type=jnp.float32)
    h_ref[...] = h
    as_ref[...] = jnp.sum(h * asv_ref[...], axis=1)[None, None, :]
    ad_ref[...] = jnp.sum(h * adv_ref[...], axis=1)[None, None, :]


def _tc_head(x, W, a_src, a_dst):
    return pl.pallas_call(
        _tc_head_body,
        grid=(G,),
        in_specs=[
            pl.BlockSpec((R, D), lambda i: (i, 0)),
            pl.BlockSpec((D, D), lambda i: (0, 0)),
            pl.BlockSpec((1, D), lambda i: (0, 0)),
            pl.BlockSpec((1, D), lambda i: (0, 0)),
        ],
        out_specs=[
            pl.BlockSpec((R, D), lambda i: (i, 0)),
            pl.BlockSpec((1, 1, R), lambda i: (i, 0, 0)),
            pl.BlockSpec((1, 1, R), lambda i: (i, 0, 0)),
        ],
        out_shape=[
            jax.ShapeDtypeStruct((NT, D), jnp.float32),
            jax.ShapeDtypeStruct((G, 1, R), jnp.float32),
            jax.ShapeDtypeStruct((G, 1, R), jnp.float32),
        ],
    )(x, W, a_src.reshape(1, D), a_dst.reshape(1, D))


def _tc_mid_body(num_ref, den_ref, b_ref, w_ref, asv_ref, adv_ref,
                 h_ref, as_ref, ad_ref):
    num = num_ref[0] + num_ref[1]
    den = den_ref[0] + den_ref[1]
    y = num / (den + 1e-16)[:, None] + b_ref[...]
    h = jnp.dot(y, w_ref[...], preferred_element_type=jnp.float32)
    h_ref[...] = h
    as_ref[...] = jnp.sum(h * asv_ref[...], axis=1)[None, None, :]
    ad_ref[...] = jnp.sum(h * adv_ref[...], axis=1)[None, None, :]


def _tc_mid(num, den, b, W, a_src, a_dst):
    return pl.pallas_call(
        _tc_mid_body,
        grid=(G,),
        in_specs=[
            pl.BlockSpec((2, R, D), lambda i: (0, i, 0)),
            pl.BlockSpec((2, R), lambda i: (0, i)),
            pl.BlockSpec((1, D), lambda i: (0, 0)),
            pl.BlockSpec((D, D), lambda i: (0, 0)),
            pl.BlockSpec((1, D), lambda i: (0, 0)),
            pl.BlockSpec((1, D), lambda i: (0, 0)),
        ],
        out_specs=[
            pl.BlockSpec((R, D), lambda i: (i, 0)),
            pl.BlockSpec((1, 1, R), lambda i: (i, 0, 0)),
            pl.BlockSpec((1, 1, R), lambda i: (i, 0, 0)),
        ],
        out_shape=[
            jax.ShapeDtypeStruct((NT, D), jnp.float32),
            jax.ShapeDtypeStruct((G, 1, R), jnp.float32),
            jax.ShapeDtypeStruct((G, 1, R), jnp.float32),
        ],
    )(num, den, b.reshape(1, D), W, a_src.reshape(1, D), a_dst.reshape(1, D))


def _tc_fin_body(num_ref, den_ref, b_ref, o_ref):
    num = num_ref[0] + num_ref[1]
    den = den_ref[0] + den_ref[1]
    y = num / (den + 1e-16)[:, None] + b_ref[...]
    o_ref[...] = jnp.maximum(y, 0.0)


def _tc_fin(num, den, b):
    return pl.pallas_call(
        _tc_fin_body,
        grid=(G,),
        in_specs=[
            pl.BlockSpec((2, R, D), lambda i: (0, i, 0)),
            pl.BlockSpec((2, R), lambda i: (0, i)),
            pl.BlockSpec((1, D), lambda i: (0, 0)),
        ],
        out_specs=pl.BlockSpec((R, D), lambda i: (i, 0)),
        out_shape=jax.ShapeDtypeStruct((NT, D), jnp.float32),
    )(num, den, b.reshape(1, D))


def _sc_edge_body(h_hbm, src_hbm, dst_hbm, asrc_hbm, adst_hbm,
                  num_out, den_out,
                  asrc_v, adst_v, sidx_v, didx_v, ex_v, rows_v,
                  num_sh, den_sh, sem):
    c = lax.axis_index("c")
    s = lax.axis_index("s")
    wid = c * NS + s

    pltpu.sync_copy(asrc_hbm, asrc_v)
    pltpu.sync_copy(adst_hbm, adst_v)

    init = (jnp.full((L,), -1e30, jnp.float32),
            jnp.full((L,), -1e30, jnp.float32))

    @pl.loop(0, NT // L, init_carry=init)
    def _maxloop(i, carry):
        ma, md = carry
        sl = pl.ds(i * L, L)
        return (jnp.maximum(ma, asrc_v[sl]), jnp.maximum(md, adst_v[sl]))

    ma, md = _maxloop

    lanes = lax.iota(jnp.int32, L)

    def _lane_max(vec):
        m = vec
        for k in (1, 2, 4, 8):
            ex_v[pl.ds(0, L)] = m
            m = jnp.maximum(m, plsc.load_gather(ex_v, [lanes ^ k]))
        return m

    csum = _lane_max(ma) + _lane_max(md)
    cub = jnp.maximum(csum, 0.2 * csum)

    zv = jnp.zeros((L,), jnp.float32)

    @pl.loop(0, B)
    def _zrows(ei):
        for j in range(D // L):
            rows_v[ei, pl.ds(j * L, L)] = zv

    for j in range(B // L):
        ex_v[pl.ds(j * L, L)] = zv

    for k in range(5):
        pltpu.sync_copy(rows_v, num_sh.at[pl.ds(s * 640 + k * B, B)])
        pltpu.sync_copy(ex_v, den_sh.at[pl.ds(s * 640 + k * B, B)])
    plsc.subcore_barrier()

    @pl.loop(0, CHUNKS)
    def _chunk(ch):
        base = wid * EPT + ch * B
        pltpu.sync_copy(src_hbm.at[pl.ds(base, B)], sidx_v)
        pltpu.sync_copy(dst_hbm.at[pl.ds(base, B)], didx_v)
        pltpu.async_copy(h_hbm.at[sidx_v], rows_v, sem).wait()
        for j in range(B // L):
            sl = pl.ds(j * L, L)
            si = sidx_v[sl]
            di = didx_v[sl]
            al = plsc.load_gather(asrc_v, [si]) + plsc.load_gather(adst_v, [di])
            al = jnp.maximum(al, 0.2 * al)
            ex_v[sl] = jnp.exp(al - cub)

        @pl.loop(0, B // L)
        def _scale(g):
            exg = ex_v[pl.ds(g * L, L)]
            for k in range(L):
                coef = exg[k]
                ei = g * L + k
                for j in range(D // L):
                    sl = pl.ds(j * L, L)
                    rows_v[ei, sl] = rows_v[ei, sl] * coef

        pltpu.sync_copy(rows_v, num_sh.at[didx_v], add=True)
        pltpu.sync_copy(ex_v, den_sh.at[didx_v], add=True)

    plsc.subcore_barrier()

    for k in range(5):
        sl = pl.ds(s * 640 + k * B, B)
        pltpu.sync_copy(num_sh.at[sl], num_out.at[c, sl])
    pltpu.sync_copy(den_sh.at[pl.ds(s * 640, 640)],
                    den_out.at[c, pl.ds(s * 640, 640)])


_sc_edge = pl.kernel(
    _sc_edge_body,
    out_type=(
        jax.ShapeDtypeStruct((NC, NT, D), jnp.float32),
        jax.ShapeDtypeStruct((NC, NT), jnp.float32),
    ),
    mesh=plsc.VectorSubcoreMesh(core_axis_name="c", subcore_axis_name="s",
                                num_cores=NC, num_subcores=NS),
    compiler_params=pltpu.CompilerParams(needs_layout_passes=False),
    scratch_types=[
        pltpu.VMEM((NT,), jnp.float32),
        pltpu.VMEM((NT,), jnp.float32),
        pltpu.VMEM((B,), jnp.int32),
        pltpu.VMEM((B,), jnp.int32),
        pltpu.VMEM((B,), jnp.float32),
        pltpu.VMEM((B, D), jnp.float32),
        pltpu.VMEM_SHARED((NT, D), jnp.float32),
        pltpu.VMEM_SHARED((NT,), jnp.float32),
        pltpu.SemaphoreType.DMA,
    ],
)


def kernel(x, e, W1, a_src1, a_dst1, b1, W2, a_src2, a_dst2, b2):
    src = e[0].astype(jnp.int32)
    dst = e[1].astype(jnp.int32)
    pad = E_PAD - src.shape[0]
    src_p = jnp.concatenate([src, jnp.full((pad,), N, jnp.int32)])
    dst_p = jnp.concatenate([dst, jnp.full((pad,), N, jnp.int32)])
    xp = jnp.pad(x, ((0, NT - N), (0, 0)))

    h1, as1, ad1 = _tc_head(xp, W1, a_src1, a_dst1)
    num1, den1 = _sc_edge(h1, src_p, dst_p, as1.reshape(-1), ad1.reshape(-1))
    h2, as2, ad2 = _tc_mid(num1, den1, b1, W2, a_src2, a_dst2)
    num2, den2 = _sc_edge(h2, src_p, dst_p, as2.reshape(-1), ad2.reshape(-1))
    return _tc_fin(num2, den2, b2)[:N]
